# baseline (speedup 1.0000x reference)
import math

import jax
import jax.numpy as jnp
from jax import lax
from jax.experimental import pallas as pl
from jax.experimental.pallas import tpu as pltpu

T = 2.0
LOG_2PI = math.log(2.0 * math.pi)
_CONTRACT = (((1,), (1,)), ((), ()))

_CHUNKS = (1024, 2048, 4096, 4096, 4096, 1024)
_OFFSETS = tuple(sum(_CHUNKS[:k]) for k in range(len(_CHUNKS)))
_AHEAD = 3


def _flow_kernel(
    thetaT_hbm,
    ctx_hbm,
    wmuT_hbm,
    bmu_ref,
    wlsT_hbm,
    bls_ref,
    out_ref,
    th_buf,
    ctx_buf,
    w_buf,
    sems,
    wsems,
):
    nchunk = len(_CHUNKS)
    d = wmuT_hbm.shape[0]

    def th_copy(c):
        off, sz = _OFFSETS[c], _CHUNKS[c]
        return pltpu.make_async_copy(
            thetaT_hbm.at[:, pl.ds(off, sz)],
            th_buf.at[:, pl.ds(off, sz)],
            sems.at[0, c],
        )

    def ctx_copy(c):
        off, sz = _OFFSETS[c], _CHUNKS[c]
        return pltpu.make_async_copy(
            ctx_hbm.at[pl.ds(off, sz), :],
            ctx_buf.at[pl.ds(off, sz), :],
            sems.at[1, c],
        )

    w_copies = (
        pltpu.make_async_copy(wmuT_hbm, w_buf.at[pl.ds(0, d)], wsems.at[0]),
        pltpu.make_async_copy(wlsT_hbm, w_buf.at[pl.ds(d, d)], wsems.at[1]),
    )
    for cp in w_copies:
        cp.start()
    for k in range(_AHEAD):
        th_copy(k).start()
        ctx_copy(k).start()
    for cp in w_copies:
        cp.wait()

    wcat = w_buf[...]
    bcat = jnp.concatenate([bmu_ref[...], bls_ref[...]], axis=0)[:, None]
    const = 0.5 * d * LOG_2PI / T

    for i in range(nchunk):
        nxt = i + _AHEAD
        if nxt < nchunk:
            th_copy(nxt).start()
            ctx_copy(nxt).start()
        th_copy(i).wait()
        ctx_copy(i).wait()
        off, sz = _OFFSETS[i], _CHUNKS[i]
        acc = (
            lax.dot_general(
                wcat,
                ctx_buf[pl.ds(off, sz), :],
                _CONTRACT,
                preferred_element_type=jnp.float32,
            )
            + bcat
        )
        mu = acc[:d]
        ls = jnp.tanh(acc[d:])
        z = (th_buf[:, pl.ds(off, sz)] - mu) * jnp.exp(-ls)
        v = z * z + 2.0 * ls
        out_ref[pl.ds(off, sz)] = (-0.5 / T) * jnp.sum(v, axis=0) - const


@jax.jit
def kernel(theta, context, W_mu, b_mu, W_ls, b_ls):
    n, d = theta.shape
    c = context.shape[-1]
    return pl.pallas_call(
        _flow_kernel,
        in_specs=[
            pl.BlockSpec(memory_space=pl.ANY),
            pl.BlockSpec(memory_space=pl.ANY),
            pl.BlockSpec(memory_space=pl.ANY),
            pl.BlockSpec((d,), lambda: (0,)),
            pl.BlockSpec(memory_space=pl.ANY),
            pl.BlockSpec((d,), lambda: (0,)),
        ],
        out_specs=pl.BlockSpec((n,), lambda: (0,)),
        out_shape=jax.ShapeDtypeStruct((n,), jnp.float32),
        scratch_shapes=[
            pltpu.VMEM((d, n), jnp.float32),
            pltpu.VMEM((n, c), jnp.float32),
            pltpu.VMEM((2 * d, c), jnp.float32),
            pltpu.SemaphoreType.DMA((2, len(_CHUNKS))),
            pltpu.SemaphoreType.DMA((2,)),
        ],
    )(theta.T, context, W_mu.T, b_mu, W_ls.T, b_ls)
